# baseline (speedup 1.0000x reference)
import functools

import jax
import jax.numpy as jnp
from jax import lax
from jax.experimental import pallas as pl
from jax.experimental.pallas import tpu as pltpu
from jax.experimental.pallas import tpu_sc as plsc

_CH = 2048
_EPS = 1e-5

_SC_ROWS = 16
_SC_LANES = 16


def _sc_item_embed(B, NT, n_work, it_hbm, eut_hbm, ei_hbm,
                   buf, red, idxm, eirows, sems, gsem):
    rows_w = B // n_work
    n_chunks = rows_w // _SC_ROWS
    wid = lax.axis_index("s") * 2 + lax.axis_index("c")
    base = wid * rows_w
    n_full = NT // _SC_LANES

    def start(chunk, b):
        pltpu.make_async_copy(
            it_hbm.at[pl.ds(base + chunk * _SC_ROWS, _SC_ROWS), :],
            buf.at[b], sems.at[b]).start()

    def wait(b):
        pltpu.make_async_copy(
            it_hbm.at[pl.ds(base, _SC_ROWS), :],
            buf.at[b], sems.at[b]).wait()

    red[pl.ds(_SC_LANES, _SC_LANES)] = jnp.zeros((_SC_LANES,), jnp.float32)
    start(0, 0)
    start(1, 1)

    def pair(g, carry):
        iota = lax.iota(jnp.int32, _SC_LANES)
        iota_f = iota.astype(jnp.float32)
        ge8 = iota >= 8
        for b in range(2):
            chunk = 2 * g + b
            wait(b)
            for r in range(_SC_ROWS):
                acc = jnp.zeros((_SC_LANES,), jnp.float32)
                for i in range(n_full):
                    w = iota_f + float(i * _SC_LANES)
                    acc = acc + buf[b, r, pl.ds(i * _SC_LANES,
                                                _SC_LANES)] * w
                v = buf[b, r, pl.ds(NT - _SC_LANES, _SC_LANES)]
                w = iota_f + float(NT - _SC_LANES)
                acc = acc + jnp.where(ge8, v, 0.0) * w
                for sh in (8, 4, 2, 1):
                    red[pl.ds(0, _SC_LANES)] = acc
                    acc = acc + red[pl.ds(sh, _SC_LANES)]
                idxm[pl.ds(r, _SC_LANES)] = acc

            @pl.when(chunk + 2 < n_chunks)
            def _():
                start(chunk + 2, b)

            idx_i = idxm[pl.ds(0, _SC_LANES)].astype(jnp.int32)
            g_cp = pltpu.make_async_copy(eut_hbm.at[idx_i], eirows, gsem)
            g_cp.start()
            g_cp.wait()
            pltpu.sync_copy(
                eirows,
                ei_hbm.at[pl.ds(base + chunk * _SC_ROWS, _SC_ROWS), :])
        return carry

    lax.fori_loop(0, n_chunks // 2, pair, 0)


def _pass1_body(uf_ref, if_ref, ut_ref, wu_ref, bu_ref, wi_ref,
                bi_ref, eit_ref, feat_ref, stats_ref):
    j = pl.program_id(0)
    bf = jnp.bfloat16
    u = jax.lax.dot_general(
        uf_ref[...].astype(bf), wu_ref[...].astype(bf),
        (((1,), (1,)), ((), ())),
        preferred_element_type=jnp.float32) + bu_ref[...]
    i = jax.lax.dot_general(
        if_ref[...].astype(bf), wi_ref[...].astype(bf),
        (((1,), (1,)), ((), ())),
        preferred_element_type=jnp.float32) + bi_ref[...]
    e_u = jax.lax.dot_general(
        ut_ref[...].astype(bf), eit_ref[...].astype(bf),
        (((1,), (0,)), ((), ())),
        preferred_element_type=jnp.float32) / 10.0
    feat = jnp.concatenate(
        [u, e_u, i, jnp.zeros((_CH, 7), jnp.float32)], axis=1)
    feat_ref[...] = feat
    s = jnp.sum(feat, axis=0, keepdims=True)
    ss = jnp.sum(feat * feat, axis=0, keepdims=True)
    part = jnp.concatenate([s, ss, jnp.zeros((6, 32), jnp.float32)], axis=0)

    @pl.when(j == 0)
    def _init():
        stats_ref[...] = part

    @pl.when(j != 0)
    def _acc():
        stats_ref[...] += part


def _pass2_body(nrows, feat_ref, ei_ref, stats_ref, gfull_ref, befull_ref,
                mask_ref, w1_ref, w1e_ref, b1_ref, w2_ref, b2_ref, w3_ref,
                b3_ref, out_ref):
    s = stats_ref[0:1, :]
    ss = stats_ref[1:2, :]
    m = s / nrows
    v = ss / nrows - m * m
    bn = mask_ref[...] > 0.5
    scale = jnp.where(bn, gfull_ref[...] * jax.lax.rsqrt(v + _EPS), 1.0)
    shift = jnp.where(bn, befull_ref[...] - m * scale, 0.0)
    y = feat_ref[...] * scale + shift
    h1 = jax.lax.dot_general(
        y, w1_ref[...], (((1,), (1,)), ((), ())),
        preferred_element_type=jnp.float32)
    h1 = h1 + jax.lax.dot_general(
        ei_ref[...], w1e_ref[...], (((1,), (1,)), ((), ())),
        preferred_element_type=jnp.float32) + b1_ref[...]
    h1 = jnp.maximum(h1, 0.0)
    h2 = jax.lax.dot_general(
        h1, w2_ref[...], (((1,), (1,)), ((), ())),
        preferred_element_type=jnp.float32) + b2_ref[...]
    h2 = jnp.maximum(h2, 0.0)
    o = jax.lax.dot_general(
        h2, w3_ref[...], (((1,), (1,)), ((), ())),
        preferred_element_type=jnp.float32) + b3_ref[...]
    out_ref[...] = jnp.maximum(o[:, 0:1], 0.0)


def kernel(user_idx, item_idx, user_feature, item_feature, user_tag, item_tag,
           Wu, bu, Wi, bi, g1, be1, g2, be2, Eut, Eit, W1, b1, W2, b2, W3, b3):
    del user_idx, item_idx
    B, DU = user_feature.shape
    NT = user_tag.shape[1]
    n_chunks = B // _CH

    info = plsc.get_sparse_core_info()
    n_work = info.num_cores * info.num_subcores
    rows_w = B // n_work
    mesh = plsc.VectorSubcoreMesh(core_axis_name="c", subcore_axis_name="s")
    Eut_p = jnp.pad(Eut, ((0, 0), (0, 128 - Eut.shape[1])))
    sc_fn = pl.kernel(
        functools.partial(_sc_item_embed, B, NT, n_work),
        mesh=mesh,
        out_type=jax.ShapeDtypeStruct((B, 128), jnp.float32),
        scratch_types=[
            pltpu.VMEM((2, _SC_ROWS, NT), jnp.float32),
            pltpu.VMEM((2 * _SC_LANES,), jnp.float32),
            pltpu.VMEM((2 * _SC_LANES,), jnp.float32),
            pltpu.VMEM((_SC_ROWS, 128), jnp.float32),
            pltpu.SemaphoreType.DMA((2,)),
            pltpu.SemaphoreType.DMA,
        ],
    )
    ei = sc_fn(item_tag, Eut_p)

    feat, stats = pl.pallas_call(
        _pass1_body,
        grid=(n_chunks,),
        in_specs=[
            pl.BlockSpec((_CH, DU), lambda j: (j, 0)),
            pl.BlockSpec((_CH, item_feature.shape[1]), lambda j: (j, 0)),
            pl.BlockSpec((_CH, NT), lambda j: (j, 0)),
            pl.BlockSpec(Wu.shape, lambda j: (0, 0)),
            pl.BlockSpec((1, 10), lambda j: (0, 0)),
            pl.BlockSpec(Wi.shape, lambda j: (0, 0)),
            pl.BlockSpec((1, 10), lambda j: (0, 0)),
            pl.BlockSpec(Eit.shape, lambda j: (0, 0)),
        ],
        out_specs=[
            pl.BlockSpec((_CH, 32), lambda j: (j, 0)),
            pl.BlockSpec((8, 32), lambda j: (0, 0)),
        ],
        out_shape=[
            jax.ShapeDtypeStruct((B, 32), jnp.float32),
            jax.ShapeDtypeStruct((8, 32), jnp.float32),
        ],
        compiler_params=pltpu.CompilerParams(
            dimension_semantics=("arbitrary",)),
    )(user_feature, item_feature, user_tag,
      Wu, bu.reshape(1, 10), Wi, bi.reshape(1, 10), Eit)

    ones5 = jnp.ones((5,), jnp.float32)
    zeros5 = jnp.zeros((5,), jnp.float32)
    pad7 = jnp.zeros((7,), jnp.float32)
    gfull = jnp.concatenate([g1, ones5, g2, pad7]).reshape(1, 32)
    befull = jnp.concatenate([be1, zeros5, be2, pad7]).reshape(1, 32)
    mask = jnp.concatenate(
        [jnp.ones((10,), jnp.float32), zeros5,
         jnp.ones((10,), jnp.float32), pad7]).reshape(1, 32)
    W1p = jnp.pad(W1[:, :25], ((0, 0), (0, 7)))
    W1e = jnp.pad(W1[:, 25:30], ((0, 0), (0, 123)))
    W3p = jnp.pad(W3, ((0, 127), (0, 0)))
    b3p = jnp.broadcast_to(b3.reshape(1, 1), (1, 128))

    out = pl.pallas_call(
        functools.partial(_pass2_body, float(B)),
        grid=(1,),
        in_specs=[
            pl.BlockSpec((B, 32), lambda j: (0, 0)),
            pl.BlockSpec((B, 128), lambda j: (0, 0)),
            pl.BlockSpec((8, 32), lambda j: (0, 0)),
            pl.BlockSpec((1, 32), lambda j: (0, 0)),
            pl.BlockSpec((1, 32), lambda j: (0, 0)),
            pl.BlockSpec((1, 32), lambda j: (0, 0)),
            pl.BlockSpec(W1p.shape, lambda j: (0, 0)),
            pl.BlockSpec(W1e.shape, lambda j: (0, 0)),
            pl.BlockSpec((1, 64), lambda j: (0, 0)),
            pl.BlockSpec(W2.shape, lambda j: (0, 0)),
            pl.BlockSpec((1, 32), lambda j: (0, 0)),
            pl.BlockSpec(W3p.shape, lambda j: (0, 0)),
            pl.BlockSpec((1, 128), lambda j: (0, 0)),
        ],
        out_specs=pl.BlockSpec((B, 1), lambda j: (0, 0)),
        out_shape=jax.ShapeDtypeStruct((B, 1), jnp.float32),
        compiler_params=pltpu.CompilerParams(
            dimension_semantics=("arbitrary",)),
    )(feat, ei, stats, gfull, befull, mask, W1p, W1e, b1.reshape(1, 64), W2,
      b2.reshape(1, 32), W3p, b3p)
    return out


# speedup vs baseline: 1.1583x; 1.1583x over previous
import functools

import jax
import jax.numpy as jnp
from jax import lax
from jax.experimental import pallas as pl
from jax.experimental.pallas import tpu as pltpu
from jax.experimental.pallas import tpu_sc as plsc

_CH = 2048
_EPS = 1e-5

_SC_ROWS = 16
_SC_LANES = 16


def _sc_item_embed(B, NT, n_work, it_hbm, eut_hbm, ei_hbm,
                   buf, red, idxm, eirows, sems, gsems, wsems):
    rows_w = B // n_work
    n_chunks = rows_w // _SC_ROWS
    wid = lax.axis_index("s") * 2 + lax.axis_index("c")
    base = wid * rows_w
    n_full = NT // _SC_LANES

    def start(chunk, b):
        pltpu.make_async_copy(
            it_hbm.at[pl.ds(base + chunk * _SC_ROWS, _SC_ROWS), :],
            buf.at[b], sems.at[b]).start()

    def wait(b):
        pltpu.make_async_copy(
            it_hbm.at[pl.ds(base, _SC_ROWS), :],
            buf.at[b], sems.at[b]).wait()

    red[pl.ds(_SC_LANES, _SC_LANES)] = jnp.zeros((_SC_LANES,), jnp.float32)
    start(0, 0)
    start(1, 1)

    def pair(g, carry):
        iota = lax.iota(jnp.int32, _SC_LANES)
        iota_f = iota.astype(jnp.float32)
        ge8 = iota >= 8
        for b in range(2):
            chunk = 2 * g + b
            wait(b)
            for r in range(_SC_ROWS):
                acc = jnp.zeros((_SC_LANES,), jnp.float32)
                for i in range(n_full):
                    w = iota_f + float(i * _SC_LANES)
                    acc = acc + buf[b, r, pl.ds(i * _SC_LANES,
                                                _SC_LANES)] * w
                v = buf[b, r, pl.ds(NT - _SC_LANES, _SC_LANES)]
                w = iota_f + float(NT - _SC_LANES)
                acc = acc + jnp.where(ge8, v, 0.0) * w
                for sh in (8, 4, 2, 1):
                    red[pl.ds(0, _SC_LANES)] = acc
                    acc = acc + red[pl.ds(sh, _SC_LANES)]
                idxm[pl.ds(r, _SC_LANES)] = acc

            @pl.when(chunk + 2 < n_chunks)
            def _():
                start(chunk + 2, b)

            idx_i = idxm[pl.ds(0, _SC_LANES)].astype(jnp.int32)

            @pl.when(g >= 1)
            def _():
                pltpu.make_async_copy(
                    eirows.at[b],
                    ei_hbm.at[pl.ds(base, _SC_ROWS), :],
                    wsems.at[b]).wait()

            pltpu.make_async_copy(
                eut_hbm.at[idx_i], eirows.at[b], gsems.at[b]).start()

        zidx = jnp.zeros((_SC_LANES,), jnp.int32)
        for b in range(2):
            chunk = 2 * g + b
            pltpu.make_async_copy(
                eut_hbm.at[zidx], eirows.at[b], gsems.at[b]).wait()
            pltpu.make_async_copy(
                eirows.at[b],
                ei_hbm.at[pl.ds(base + chunk * _SC_ROWS, _SC_ROWS), :],
                wsems.at[b]).start()
        return carry

    lax.fori_loop(0, n_chunks // 2, pair, 0)
    for b in range(2):
        pltpu.make_async_copy(
            eirows.at[b],
            ei_hbm.at[pl.ds(base, _SC_ROWS), :],
            wsems.at[b]).wait()


def _pass1_body(uf_ref, if_ref, ut_ref, wu_ref, bu_ref, wi_ref,
                bi_ref, eit_ref, feat_ref, stats_ref):
    j = pl.program_id(0)
    bf = jnp.bfloat16
    u = jax.lax.dot_general(
        uf_ref[...].astype(bf), wu_ref[...].astype(bf),
        (((1,), (1,)), ((), ())),
        preferred_element_type=jnp.float32) + bu_ref[...]
    i = jax.lax.dot_general(
        if_ref[...].astype(bf), wi_ref[...].astype(bf),
        (((1,), (1,)), ((), ())),
        preferred_element_type=jnp.float32) + bi_ref[...]
    e_u = jax.lax.dot_general(
        ut_ref[...].astype(bf), eit_ref[...].astype(bf),
        (((1,), (0,)), ((), ())),
        preferred_element_type=jnp.float32) / 10.0
    feat = jnp.concatenate(
        [u, e_u, i, jnp.zeros((_CH, 7), jnp.float32)], axis=1)
    feat_ref[...] = feat
    s = jnp.sum(feat, axis=0, keepdims=True)
    ss = jnp.sum(feat * feat, axis=0, keepdims=True)
    part = jnp.concatenate([s, ss, jnp.zeros((6, 32), jnp.float32)], axis=0)

    @pl.when(j == 0)
    def _init():
        stats_ref[...] = part

    @pl.when(j != 0)
    def _acc():
        stats_ref[...] += part


def _pass2_body(nrows, feat_ref, ei_ref, stats_ref, gfull_ref, befull_ref,
                mask_ref, w1_ref, w1e_ref, b1_ref, w2_ref, b2_ref, w3_ref,
                b3_ref, out_ref):
    s = stats_ref[0:1, :]
    ss = stats_ref[1:2, :]
    m = s / nrows
    v = ss / nrows - m * m
    bn = mask_ref[...] > 0.5
    scale = jnp.where(bn, gfull_ref[...] * jax.lax.rsqrt(v + _EPS), 1.0)
    shift = jnp.where(bn, befull_ref[...] - m * scale, 0.0)
    y = feat_ref[...] * scale + shift
    h1 = jax.lax.dot_general(
        y, w1_ref[...], (((1,), (1,)), ((), ())),
        preferred_element_type=jnp.float32)
    h1 = h1 + jax.lax.dot_general(
        ei_ref[...], w1e_ref[...], (((1,), (1,)), ((), ())),
        preferred_element_type=jnp.float32) + b1_ref[...]
    h1 = jnp.maximum(h1, 0.0)
    h2 = jax.lax.dot_general(
        h1, w2_ref[...], (((1,), (1,)), ((), ())),
        preferred_element_type=jnp.float32) + b2_ref[...]
    h2 = jnp.maximum(h2, 0.0)
    o = jax.lax.dot_general(
        h2, w3_ref[...], (((1,), (1,)), ((), ())),
        preferred_element_type=jnp.float32) + b3_ref[...]
    out_ref[...] = jnp.maximum(o[:, 0:1], 0.0)


def kernel(user_idx, item_idx, user_feature, item_feature, user_tag, item_tag,
           Wu, bu, Wi, bi, g1, be1, g2, be2, Eut, Eit, W1, b1, W2, b2, W3, b3):
    del user_idx, item_idx
    B, DU = user_feature.shape
    NT = user_tag.shape[1]
    n_chunks = B // _CH

    info = plsc.get_sparse_core_info()
    n_work = info.num_cores * info.num_subcores
    rows_w = B // n_work
    mesh = plsc.VectorSubcoreMesh(core_axis_name="c", subcore_axis_name="s")
    Eut_p = jnp.pad(Eut, ((0, 0), (0, 128 - Eut.shape[1])))
    sc_fn = pl.kernel(
        functools.partial(_sc_item_embed, B, NT, n_work),
        mesh=mesh,
        out_type=jax.ShapeDtypeStruct((B, 128), jnp.float32),
        scratch_types=[
            pltpu.VMEM((2, _SC_ROWS, NT), jnp.float32),
            pltpu.VMEM((2 * _SC_LANES,), jnp.float32),
            pltpu.VMEM((2 * _SC_LANES,), jnp.float32),
            pltpu.VMEM((2, _SC_ROWS, 128), jnp.float32),
            pltpu.SemaphoreType.DMA((2,)),
            pltpu.SemaphoreType.DMA((2,)),
            pltpu.SemaphoreType.DMA((2,)),
        ],
    )
    ei = sc_fn(item_tag, Eut_p)

    feat, stats = pl.pallas_call(
        _pass1_body,
        grid=(n_chunks,),
        in_specs=[
            pl.BlockSpec((_CH, DU), lambda j: (j, 0)),
            pl.BlockSpec((_CH, item_feature.shape[1]), lambda j: (j, 0)),
            pl.BlockSpec((_CH, NT), lambda j: (j, 0)),
            pl.BlockSpec(Wu.shape, lambda j: (0, 0)),
            pl.BlockSpec((1, 10), lambda j: (0, 0)),
            pl.BlockSpec(Wi.shape, lambda j: (0, 0)),
            pl.BlockSpec((1, 10), lambda j: (0, 0)),
            pl.BlockSpec(Eit.shape, lambda j: (0, 0)),
        ],
        out_specs=[
            pl.BlockSpec((_CH, 32), lambda j: (j, 0)),
            pl.BlockSpec((8, 32), lambda j: (0, 0)),
        ],
        out_shape=[
            jax.ShapeDtypeStruct((B, 32), jnp.float32),
            jax.ShapeDtypeStruct((8, 32), jnp.float32),
        ],
        compiler_params=pltpu.CompilerParams(
            dimension_semantics=("arbitrary",)),
    )(user_feature, item_feature, user_tag,
      Wu, bu.reshape(1, 10), Wi, bi.reshape(1, 10), Eit)

    ones5 = jnp.ones((5,), jnp.float32)
    zeros5 = jnp.zeros((5,), jnp.float32)
    pad7 = jnp.zeros((7,), jnp.float32)
    gfull = jnp.concatenate([g1, ones5, g2, pad7]).reshape(1, 32)
    befull = jnp.concatenate([be1, zeros5, be2, pad7]).reshape(1, 32)
    mask = jnp.concatenate(
        [jnp.ones((10,), jnp.float32), zeros5,
         jnp.ones((10,), jnp.float32), pad7]).reshape(1, 32)
    W1p = jnp.pad(W1[:, :25], ((0, 0), (0, 7)))
    W1e = jnp.pad(W1[:, 25:30], ((0, 0), (0, 123)))
    W3p = jnp.pad(W3, ((0, 127), (0, 0)))
    b3p = jnp.broadcast_to(b3.reshape(1, 1), (1, 128))

    out = pl.pallas_call(
        functools.partial(_pass2_body, float(B)),
        grid=(1,),
        in_specs=[
            pl.BlockSpec((B, 32), lambda j: (0, 0)),
            pl.BlockSpec((B, 128), lambda j: (0, 0)),
            pl.BlockSpec((8, 32), lambda j: (0, 0)),
            pl.BlockSpec((1, 32), lambda j: (0, 0)),
            pl.BlockSpec((1, 32), lambda j: (0, 0)),
            pl.BlockSpec((1, 32), lambda j: (0, 0)),
            pl.BlockSpec(W1p.shape, lambda j: (0, 0)),
            pl.BlockSpec(W1e.shape, lambda j: (0, 0)),
            pl.BlockSpec((1, 64), lambda j: (0, 0)),
            pl.BlockSpec(W2.shape, lambda j: (0, 0)),
            pl.BlockSpec((1, 32), lambda j: (0, 0)),
            pl.BlockSpec(W3p.shape, lambda j: (0, 0)),
            pl.BlockSpec((1, 128), lambda j: (0, 0)),
        ],
        out_specs=pl.BlockSpec((B, 1), lambda j: (0, 0)),
        out_shape=jax.ShapeDtypeStruct((B, 1), jnp.float32),
        compiler_params=pltpu.CompilerParams(
            dimension_semantics=("arbitrary",)),
    )(feat, ei, stats, gfull, befull, mask, W1p, W1e, b1.reshape(1, 64), W2,
      b2.reshape(1, 32), W3p, b3p)
    return out
